# baseline (speedup 1.0000x reference)
import jax
import jax.numpy as jnp
from jax import lax
from jax.experimental import pallas as pl
from jax.experimental.pallas import tpu as pltpu
from jax.experimental.pallas import tpu_sc as plsc

N_ROWS = 8192
N_CH = 2048
NUM_CORES = 2
NUM_SUBCORES = 16
NUM_WORKERS = NUM_CORES * NUM_SUBCORES
ROWS_PER_WORKER = N_ROWS // NUM_WORKERS
RB = 8
NUM_BLOCKS = ROWS_PER_WORKER // RB
LANES = 16
CHUNKS = N_CH // LANES


def _body(x_hbm, perm_hbm, out_hbm, perm_v, in_v, out_v):
    wid = lax.axis_index("s") * NUM_CORES + lax.axis_index("c")
    base = wid * ROWS_PER_WORKER * N_CH

    pltpu.sync_copy(perm_hbm, perm_v)

    def block(b, _):
        elem0 = base + b * (RB * N_CH)
        pltpu.sync_copy(x_hbm.at[pl.ds(elem0, RB * N_CH)], in_v)

        def row(r, _):
            roff = jnp.full((LANES,), r * N_CH, jnp.int32)

            def chunk(j, _):
                col = j * LANES
                pc = perm_v[pl.ds(col, LANES)]
                v = plsc.load_gather(in_v, [roff + pc])
                out_v[pl.ds(r * N_CH + col, LANES)] = v
                return 0

            lax.fori_loop(0, CHUNKS, chunk, 0, unroll=8)
            return 0

        lax.fori_loop(0, RB, row, 0)
        pltpu.sync_copy(out_v, out_hbm.at[pl.ds(elem0, RB * N_CH)])
        return 0

    lax.fori_loop(0, NUM_BLOCKS, block, 0)


@jax.jit
def kernel(x, perm):
    mesh = plsc.VectorSubcoreMesh(core_axis_name="c", subcore_axis_name="s")
    out_flat = pl.kernel(
        _body,
        out_type=jax.ShapeDtypeStruct((N_ROWS * N_CH,), jnp.float32),
        mesh=mesh,
        compiler_params=pltpu.CompilerParams(needs_layout_passes=False),
        scratch_types=[
            pltpu.VMEM((N_CH,), jnp.int32),
            pltpu.VMEM((RB * N_CH,), jnp.float32),
            pltpu.VMEM((RB * N_CH,), jnp.float32),
        ],
    )(x.reshape(-1), perm)
    return out_flat.reshape(N_ROWS, N_CH)


# speedup vs baseline: 1.7854x; 1.7854x over previous
import jax
import jax.numpy as jnp
from jax import lax
from jax.experimental import pallas as pl
from jax.experimental.pallas import tpu as pltpu
from jax.experimental.pallas import tpu_sc as plsc

N_ROWS = 8192
N_CH = 2048
NUM_CORES = 2
NUM_SUBCORES = 16
NUM_WORKERS = NUM_CORES * NUM_SUBCORES
ROWS_PER_WORKER = N_ROWS // NUM_WORKERS
RB = 8
BLK = RB * N_CH
NUM_BLOCKS = ROWS_PER_WORKER // RB
LANES = 16
CHUNKS = N_CH // LANES


def _body(x_hbm, perm_hbm, out_hbm, perm_v, in0, in1, out0, out1,
          sin0, sin1, sout0, sout1):
    wid = lax.axis_index("s") * NUM_CORES + lax.axis_index("c")
    base = wid * ROWS_PER_WORKER * N_CH

    ins = [in0, in1]
    outs = [out0, out1]
    sins = [sin0, sin1]
    souts = [sout0, sout1]

    pltpu.sync_copy(perm_hbm, perm_v)

    def in_start(b, k):
        pltpu.async_copy(x_hbm.at[pl.ds(base + b * BLK, BLK)], ins[k], sins[k])

    def in_wait(k):
        pltpu.make_async_copy(x_hbm.at[pl.ds(base, BLK)], ins[k], sins[k]).wait()

    def out_start(b, k):
        pltpu.async_copy(outs[k], out_hbm.at[pl.ds(base + b * BLK, BLK)], souts[k])

    def out_wait(k):
        pltpu.make_async_copy(outs[k], out_hbm.at[pl.ds(base, BLK)], souts[k]).wait()

    def permute_block(in_buf, out_buf):
        def chunk(j, _):
            col = j * LANES
            pc = perm_v[pl.ds(col, LANES)]
            for r in range(RB):
                v = plsc.load_gather(in_buf, [pc + (r * N_CH)])
                out_buf[pl.ds(col + r * N_CH, LANES)] = v
            return 0

        lax.fori_loop(0, CHUNKS, chunk, 0, unroll=2)

    in_start(0, 0)

    def outer(bb, _):
        for k in range(2):
            b = bb * 2 + k
            nxt = (k + 1) % 2

            @pl.when(b + 1 < NUM_BLOCKS)
            def _():
                in_start(b + 1, nxt)

            in_wait(k)

            @pl.when(b >= 2)
            def _():
                out_wait(k)

            permute_block(ins[k], outs[k])
            out_start(b, k)
        return 0

    lax.fori_loop(0, NUM_BLOCKS // 2, outer, 0)
    out_wait(0)
    out_wait(1)


@jax.jit
def kernel(x, perm):
    mesh = plsc.VectorSubcoreMesh(core_axis_name="c", subcore_axis_name="s")
    out_flat = pl.kernel(
        _body,
        out_type=jax.ShapeDtypeStruct((N_ROWS * N_CH,), jnp.float32),
        mesh=mesh,
        compiler_params=pltpu.CompilerParams(needs_layout_passes=False),
        scratch_types=[
            pltpu.VMEM((N_CH,), jnp.int32),
            pltpu.VMEM((BLK,), jnp.float32),
            pltpu.VMEM((BLK,), jnp.float32),
            pltpu.VMEM((BLK,), jnp.float32),
            pltpu.VMEM((BLK,), jnp.float32),
            pltpu.SemaphoreType.DMA,
            pltpu.SemaphoreType.DMA,
            pltpu.SemaphoreType.DMA,
            pltpu.SemaphoreType.DMA,
        ],
    )(x.reshape(-1), perm)
    return out_flat.reshape(N_ROWS, N_CH)


# speedup vs baseline: 2.7986x; 1.5675x over previous
import jax
import jax.numpy as jnp
from jax import lax
from jax.experimental import pallas as pl
from jax.experimental.pallas import tpu as pltpu
from jax.experimental.pallas import tpu_sc as plsc

N_ROWS = 8192
N_CH = 2048
NUM_CORES = 2
NUM_SUBCORES = 16
NUM_WORKERS = NUM_CORES * NUM_SUBCORES
ROWS_PER_WORKER = N_ROWS // NUM_WORKERS
RB = 8
NUM_BLOCKS = ROWS_PER_WORKER // RB
LANES = 16
CHUNKS = N_CH // LANES


def _body(x_hbm, perm_hbm, out_hbm, perm_v, in0, in1, out0, out1,
          sin0, sin1, sout0, sout1):
    wid = lax.axis_index("s") * NUM_CORES + lax.axis_index("c")
    base = wid * ROWS_PER_WORKER

    ins = [in0, in1]
    outs = [out0, out1]
    sins = [sin0, sin1]
    souts = [sout0, sout1]

    pltpu.sync_copy(perm_hbm, perm_v)

    def in_start(b, k):
        pltpu.async_copy(x_hbm.at[pl.ds(base + b * RB, RB)], ins[k], sins[k])

    def in_wait(k):
        pltpu.make_async_copy(x_hbm.at[pl.ds(base, RB)], ins[k], sins[k]).wait()

    def out_start(b, k):
        pltpu.async_copy(outs[k], out_hbm.at[pl.ds(base + b * RB, RB)], souts[k])

    def out_wait(k):
        pltpu.make_async_copy(outs[k], out_hbm.at[pl.ds(base, RB)], souts[k]).wait()

    def permute_block(in_buf, out_buf):
        def chunk(j, _):
            col = j * LANES
            pc = perm_v[pl.ds(col, LANES)]
            for r in range(RB):
                ridx = jnp.full((LANES,), r, jnp.int32)
                v = plsc.load_gather(in_buf, [ridx, pc])
                out_buf[r, pl.ds(col, LANES)] = v
            return 0

        lax.fori_loop(0, CHUNKS, chunk, 0, unroll=2)

    in_start(0, 0)

    def outer(bb, _):
        for k in range(2):
            b = bb * 2 + k
            nxt = (k + 1) % 2

            @pl.when(b + 1 < NUM_BLOCKS)
            def _():
                in_start(b + 1, nxt)

            in_wait(k)

            @pl.when(b >= 2)
            def _():
                out_wait(k)

            permute_block(ins[k], outs[k])
            out_start(b, k)
        return 0

    lax.fori_loop(0, NUM_BLOCKS // 2, outer, 0)
    out_wait(0)
    out_wait(1)


@jax.jit
def kernel(x, perm):
    mesh = plsc.VectorSubcoreMesh(core_axis_name="c", subcore_axis_name="s")
    return pl.kernel(
        _body,
        out_type=jax.ShapeDtypeStruct((N_ROWS, N_CH), jnp.float32),
        mesh=mesh,
        compiler_params=pltpu.CompilerParams(needs_layout_passes=False),
        scratch_types=[
            pltpu.VMEM((N_CH,), jnp.int32),
            pltpu.VMEM((RB, N_CH), jnp.float32),
            pltpu.VMEM((RB, N_CH), jnp.float32),
            pltpu.VMEM((RB, N_CH), jnp.float32),
            pltpu.VMEM((RB, N_CH), jnp.float32),
            pltpu.SemaphoreType.DMA,
            pltpu.SemaphoreType.DMA,
            pltpu.SemaphoreType.DMA,
            pltpu.SemaphoreType.DMA,
        ],
    )(x, perm)


# speedup vs baseline: 7.6635x; 2.7383x over previous
import jax
import jax.numpy as jnp
from jax import lax
from jax.experimental import pallas as pl
from jax.experimental.pallas import tpu as pltpu
from jax.experimental.pallas import tpu_sc as plsc

N_ROWS = 8192
N_CH = 2048
NUM_CORES = 2
NUM_SUBCORES = 16
NUM_WORKERS = NUM_CORES * NUM_SUBCORES
ROWS_PER_WORKER = N_ROWS // NUM_WORKERS
RB = 8
NUM_BLOCKS = ROWS_PER_WORKER // RB
LANES = 16
CHUNKS = N_CH // LANES


def _body(x_hbm, perm_hbm, out_hbm, perm_v, in0, in1, out0, out1,
          sin0, sin1, sout0, sout1):
    wid = lax.axis_index("s") * NUM_CORES + lax.axis_index("c")
    base = wid * ROWS_PER_WORKER

    ins = [in0, in1]
    outs = [out0, out1]
    sins = [sin0, sin1]
    souts = [sout0, sout1]

    pltpu.sync_copy(perm_hbm, perm_v)

    def in_start(b, k):
        pltpu.async_copy(x_hbm.at[pl.ds(base + b * RB, RB)], ins[k], sins[k])

    def in_wait(k):
        pltpu.make_async_copy(x_hbm.at[pl.ds(base, RB)], ins[k], sins[k]).wait()

    def out_start(b, k):
        pltpu.async_copy(outs[k], out_hbm.at[pl.ds(base + b * RB, RB)], souts[k])

    def out_wait(k):
        pltpu.make_async_copy(outs[k], out_hbm.at[pl.ds(base, RB)], souts[k]).wait()

    ridxs = [jnp.full((LANES,), r, jnp.int32) for r in range(RB)]

    def permute_block(in_buf, out_buf):
        @plsc.parallel_loop(0, N_CH, LANES, unroll=4)
        def _chunk(col):
            pc = perm_v[pl.ds(col, LANES)]
            for r in range(RB):
                v = plsc.load_gather(in_buf, [ridxs[r], pc])
                out_buf[r, pl.ds(col, LANES)] = v

    in_start(0, 0)

    def outer(bb, _):
        for k in range(2):
            b = bb * 2 + k
            nxt = (k + 1) % 2

            @pl.when(b + 1 < NUM_BLOCKS)
            def _():
                in_start(b + 1, nxt)

            in_wait(k)

            @pl.when(b >= 2)
            def _():
                out_wait(k)

            permute_block(ins[k], outs[k])
            out_start(b, k)
        return 0

    lax.fori_loop(0, NUM_BLOCKS // 2, outer, 0)
    out_wait(0)
    out_wait(1)


@jax.jit
def kernel(x, perm):
    mesh = plsc.VectorSubcoreMesh(core_axis_name="c", subcore_axis_name="s")
    return pl.kernel(
        _body,
        out_type=jax.ShapeDtypeStruct((N_ROWS, N_CH), jnp.float32),
        mesh=mesh,
        compiler_params=pltpu.CompilerParams(needs_layout_passes=False),
        scratch_types=[
            pltpu.VMEM((N_CH,), jnp.int32),
            pltpu.VMEM((RB, N_CH), jnp.float32),
            pltpu.VMEM((RB, N_CH), jnp.float32),
            pltpu.VMEM((RB, N_CH), jnp.float32),
            pltpu.VMEM((RB, N_CH), jnp.float32),
            pltpu.SemaphoreType.DMA,
            pltpu.SemaphoreType.DMA,
            pltpu.SemaphoreType.DMA,
            pltpu.SemaphoreType.DMA,
        ],
    )(x, perm)


# speedup vs baseline: 7.6901x; 1.0035x over previous
import jax
import jax.numpy as jnp
from jax import lax
from jax.experimental import pallas as pl
from jax.experimental.pallas import tpu as pltpu
from jax.experimental.pallas import tpu_sc as plsc

N_ROWS = 8192
N_CH = 2048
NUM_CORES = 2
NUM_SUBCORES = 16
NUM_WORKERS = NUM_CORES * NUM_SUBCORES
ROWS_PER_WORKER = N_ROWS // NUM_WORKERS
RB = 8
NUM_BLOCKS = ROWS_PER_WORKER // RB
LANES = 16
CHUNKS = N_CH // LANES


def _body(x_hbm, perm_hbm, out_hbm, perm_v, in0, in1, out0, out1,
          sin0, sin1, sout0, sout1):
    wid = lax.axis_index("s") * NUM_CORES + lax.axis_index("c")
    base = wid * ROWS_PER_WORKER

    ins = [in0, in1]
    outs = [out0, out1]
    sins = [sin0, sin1]
    souts = [sout0, sout1]

    pltpu.sync_copy(perm_hbm, perm_v)

    def in_start(b, k):
        pltpu.async_copy(x_hbm.at[pl.ds(base + b * RB, RB)], ins[k], sins[k])

    def in_wait(k):
        pltpu.make_async_copy(x_hbm.at[pl.ds(base, RB)], ins[k], sins[k]).wait()

    def out_start(b, k):
        pltpu.async_copy(outs[k], out_hbm.at[pl.ds(base + b * RB, RB)], souts[k])

    def out_wait(k):
        pltpu.make_async_copy(outs[k], out_hbm.at[pl.ds(base, RB)], souts[k]).wait()

    ridxs = [jnp.full((LANES,), r, jnp.int32) for r in range(RB)]

    def permute_block(in_buf, out_buf):
        @plsc.parallel_loop(0, N_CH, LANES, unroll=8)
        def _chunk(col):
            pc = perm_v[pl.ds(col, LANES)]
            for r in range(RB):
                v = plsc.load_gather(in_buf, [ridxs[r], pc])
                out_buf[r, pl.ds(col, LANES)] = v

    in_start(0, 0)

    def outer(bb, _):
        for k in range(2):
            b = bb * 2 + k
            nxt = (k + 1) % 2

            @pl.when(b + 1 < NUM_BLOCKS)
            def _():
                in_start(b + 1, nxt)

            in_wait(k)

            @pl.when(b >= 2)
            def _():
                out_wait(k)

            permute_block(ins[k], outs[k])
            out_start(b, k)
        return 0

    lax.fori_loop(0, NUM_BLOCKS // 2, outer, 0)
    out_wait(0)
    out_wait(1)


@jax.jit
def kernel(x, perm):
    mesh = plsc.VectorSubcoreMesh(core_axis_name="c", subcore_axis_name="s")
    return pl.kernel(
        _body,
        out_type=jax.ShapeDtypeStruct((N_ROWS, N_CH), jnp.float32),
        mesh=mesh,
        compiler_params=pltpu.CompilerParams(needs_layout_passes=False),
        scratch_types=[
            pltpu.VMEM((N_CH,), jnp.int32),
            pltpu.VMEM((RB, N_CH), jnp.float32),
            pltpu.VMEM((RB, N_CH), jnp.float32),
            pltpu.VMEM((RB, N_CH), jnp.float32),
            pltpu.VMEM((RB, N_CH), jnp.float32),
            pltpu.SemaphoreType.DMA,
            pltpu.SemaphoreType.DMA,
            pltpu.SemaphoreType.DMA,
            pltpu.SemaphoreType.DMA,
        ],
    )(x, perm)


# speedup vs baseline: 7.8734x; 1.0238x over previous
import jax
import jax.numpy as jnp
from jax import lax
from jax.experimental import pallas as pl
from jax.experimental.pallas import tpu as pltpu
from jax.experimental.pallas import tpu_sc as plsc

N_ROWS = 8192
N_CH = 2048
NUM_CORES = 2
NUM_SUBCORES = 16
NUM_WORKERS = NUM_CORES * NUM_SUBCORES
ROWS_PER_WORKER = N_ROWS // NUM_WORKERS
RB = 8
NUM_BLOCKS = ROWS_PER_WORKER // RB
LANES = 16
CHUNKS = N_CH // LANES


def _body(x_hbm, perm_hbm, out_hbm, perm_v, in0, in1, in2, out0, out1, out2,
          sin0, sin1, sin2, sout0, sout1, sout2):
    wid = lax.axis_index("s") * NUM_CORES + lax.axis_index("c")
    base = wid * ROWS_PER_WORKER

    ins = [in0, in1, in2]
    outs = [out0, out1, out2]
    sins = [sin0, sin1, sin2]
    souts = [sout0, sout1, sout2]

    pltpu.sync_copy(perm_hbm, perm_v)

    def in_start(b, k):
        pltpu.async_copy(x_hbm.at[pl.ds(base + b * RB, RB)], ins[k], sins[k])

    def in_wait(k):
        pltpu.make_async_copy(x_hbm.at[pl.ds(base, RB)], ins[k], sins[k]).wait()

    def out_start(b, k):
        pltpu.async_copy(outs[k], out_hbm.at[pl.ds(base + b * RB, RB)], souts[k])

    def out_wait(k):
        pltpu.make_async_copy(outs[k], out_hbm.at[pl.ds(base, RB)], souts[k]).wait()

    ridxs = [jnp.full((LANES,), r, jnp.int32) for r in range(RB)]

    def permute_block(in_buf, out_buf):
        @plsc.parallel_loop(0, N_CH, LANES, unroll=8)
        def _chunk(col):
            pc = perm_v[pl.ds(col, LANES)]
            for r in range(RB):
                v = plsc.load_gather(in_buf, [ridxs[r], pc])
                out_buf[r, pl.ds(col, LANES)] = v

    in_start(0, 0)
    in_start(1, 1)

    def outer(bb, _):
        for k in range(3):
            b = bb * 3 + k

            @pl.when(b + 2 < NUM_BLOCKS)
            def _():
                in_start(b + 2, (k + 2) % 3)

            in_wait(k)

            @pl.when(b >= 3)
            def _():
                out_wait(k)

            permute_block(ins[k], outs[k])
            out_start(b, k)
        return 0

    lax.fori_loop(0, NUM_BLOCKS // 3, outer, 0)

    for b, k in ((30, 0), (31, 1)):
        in_wait(k)
        out_wait(k)
        permute_block(ins[k], outs[k])
        out_start(b, k)
    out_wait(2)
    out_wait(0)
    out_wait(1)


@jax.jit
def kernel(x, perm):
    mesh = plsc.VectorSubcoreMesh(core_axis_name="c", subcore_axis_name="s")
    return pl.kernel(
        _body,
        out_type=jax.ShapeDtypeStruct((N_ROWS, N_CH), jnp.float32),
        mesh=mesh,
        compiler_params=pltpu.CompilerParams(needs_layout_passes=False),
        scratch_types=[
            pltpu.VMEM((N_CH,), jnp.int32),
            pltpu.VMEM((RB, N_CH), jnp.float32),
            pltpu.VMEM((RB, N_CH), jnp.float32),
            pltpu.VMEM((RB, N_CH), jnp.float32),
            pltpu.VMEM((RB, N_CH), jnp.float32),
            pltpu.VMEM((RB, N_CH), jnp.float32),
            pltpu.VMEM((RB, N_CH), jnp.float32),
            pltpu.SemaphoreType.DMA,
            pltpu.SemaphoreType.DMA,
            pltpu.SemaphoreType.DMA,
            pltpu.SemaphoreType.DMA,
            pltpu.SemaphoreType.DMA,
            pltpu.SemaphoreType.DMA,
        ],
    )(x, perm)
